# baseline (speedup 1.0000x reference)
import functools

import jax
import jax.numpy as jnp
from jax import lax
from jax.experimental import pallas as pl
from jax.experimental.pallas import tpu as pltpu
from jax.experimental.pallas import tpu_sc as plsc

N = 10000
E = 160000
N_PAD = 10240
NUM_CORES = 2
NUM_SUBCORES = 16
NW = NUM_CORES * NUM_SUBCORES
CHUNK = 64
NCHUNK = 80
EPW = NCHUNK * CHUNK
E_PAD = NW * EPW
NPAIR = NCHUNK // 2
GC = 8
NGROUP = NCHUNK // GC
ROWS_PER_TILE = N_PAD // NUM_SUBCORES


@functools.cache
def _make_agg(H: int):
    mesh = plsc.VectorSubcoreMesh(core_axis_name="c", subcore_axis_name="s")

    @functools.partial(
        pl.kernel,
        out_type=jax.ShapeDtypeStruct((NUM_CORES, N_PAD, H), jnp.float32),
        mesh=mesh,
        compiler_params=pltpu.CompilerParams(use_tc_tiling_on_sc=True),
        scratch_types=[
            [pltpu.VMEM((GC, CHUNK), jnp.int32)] * 2,
            [pltpu.VMEM((GC, CHUNK), jnp.int32)] * 2,
            [pltpu.VMEM((CHUNK, H), jnp.float32)] * 4,
            [pltpu.SemaphoreType.DMA] * 4,
            [pltpu.SemaphoreType.DMA] * 4,
            [pltpu.SemaphoreType.DMA] * 4,
            pltpu.VMEM_SHARED((N_PAD, H), jnp.float32),
        ],
    )
    def agg(hw_hbm, src_hbm, dst_hbm, zeros_hbm, out_hbm,
            isrc, idst, gb, gsem, ssem, isem, acc):
        cid = lax.axis_index("c")
        sid = lax.axis_index("s")
        wid = cid * NUM_SUBCORES + sid
        base = sid * ROWS_PER_TILE
        pltpu.sync_copy(zeros_hbm, acc.at[pl.ds(base, ROWS_PER_TILE)])

        def idx_fetch(g, b):
            return (
                pltpu.make_async_copy(src_hbm.at[wid, pl.ds(g * GC, GC)],
                                      isrc[b], isem[2 * b]),
                pltpu.make_async_copy(dst_hbm.at[wid, pl.ds(g * GC, GC)],
                                      idst[b], isem[2 * b + 1]),
            )

        def gather(b, row, s):
            return pltpu.make_async_copy(hw_hbm.at[idst[b].at[row]], gb[s],
                                         gsem[s])

        def scatter(b, row, s):
            return pltpu.make_async_copy(gb[s], acc.at[isrc[b].at[row]],
                                         ssem[s])

        plsc.subcore_barrier()

        for d in idx_fetch(0, 0):
            d.start()

        def round_(r, t, gg):
            p = t % 2
            g0, g1 = 2 * p, 2 * p + 1
            s0, s1 = 2 - 2 * p, 3 - 2 * p
            b = 0 if t < 4 else 1
            rows = (2 * (t % 4), 2 * (t % 4) + 1)
            pb = 0 if (t - 1) % 8 < 4 else 1
            prows = (2 * ((t - 1) % 4), 2 * ((t - 1) % 4) + 1)
            db = 0 if (t - 2) % 8 < 4 else 1
            drows = (2 * ((t - 2) % 4), 2 * ((t - 2) % 4) + 1)

            @pl.when(r >= 2)
            def _():
                scatter(db, drows[0], g0).wait()
                scatter(db, drows[1], g1).wait()

            if t == 0:
                idx_fetch(2 * gg, 0)[0].wait()
                idx_fetch(2 * gg, 0)[1].wait()
            if t == 4:
                idx_fetch(2 * gg + 1, 1)[0].wait()
                idx_fetch(2 * gg + 1, 1)[1].wait()

            if t == 2:
                for d in idx_fetch(2 * gg + 1, 1):
                    d.start()
            if t == 6:
                @pl.when(2 * gg + 2 < NGROUP)
                def _():
                    for d in idx_fetch(2 * gg + 2, 0):
                        d.start()

            gather(b, rows[0], g0).start()
            gather(b, rows[1], g1).start()

            @pl.when(r >= 1)
            def _():
                gather(pb, prows[0], s0).wait()
                scatter(pb, prows[0], s0).start(add=True)
                gather(pb, prows[1], s1).wait()
                scatter(pb, prows[1], s1).start(add=True)

        def body(gg, carry):
            for t in range(8):
                round_(8 * gg + t, t, gg)
            return carry

        lax.fori_loop(0, NPAIR // 8, body, 0)
        scatter(1, 4, 0).wait()
        scatter(1, 5, 1).wait()
        gather(1, 6, 2).wait()
        scatter(1, 6, 2).start(add=True)
        gather(1, 7, 3).wait()
        scatter(1, 7, 3).start(add=True)
        scatter(1, 6, 2).wait()
        scatter(1, 7, 3).wait()

        plsc.subcore_barrier()
        pltpu.sync_copy(acc.at[pl.ds(base, ROWS_PER_TILE)],
                        out_hbm.at[cid, pl.ds(base, ROWS_PER_TILE)])

    return agg


_BM = 2000


def _mm_body(x_ref, w_ref, o_ref):
    o_ref[...] = jnp.dot(x_ref[...], w_ref[...],
                         preferred_element_type=jnp.float32)


def _mm(x, W):
    M, K = x.shape
    H = W.shape[1]
    return pl.pallas_call(
        _mm_body,
        grid=(M // _BM,),
        in_specs=[pl.BlockSpec((_BM, K), lambda i: (i, 0)),
                  pl.BlockSpec((K, H), lambda i: (0, 0))],
        out_specs=pl.BlockSpec((_BM, H), lambda i: (i, 0)),
        out_shape=jax.ShapeDtypeStruct((M, H), jnp.float32),
    )(x, W)


def _elu(z):
    return jnp.where(z > 0, z, jnp.exp(z) - 1.0)


def _elu_mm_body(zp_ref, w_ref, o_ref):
    h = _elu(zp_ref[0] + zp_ref[1])
    o_ref[...] = jnp.dot(h, w_ref[...], preferred_element_type=jnp.float32)


def _elu_mm(zp, W):
    K = zp.shape[2]
    H = W.shape[1]
    return pl.pallas_call(
        _elu_mm_body,
        grid=(N // _BM,),
        in_specs=[pl.BlockSpec((NUM_CORES, _BM, K), lambda i: (0, i, 0)),
                  pl.BlockSpec((K, H), lambda i: (0, 0))],
        out_specs=pl.BlockSpec((_BM, H), lambda i: (i, 0)),
        out_shape=jax.ShapeDtypeStruct((N, H), jnp.float32),
    )(zp, W)


def _final_body(zp_ref, o_ref):
    nc = o_ref.shape[1]
    h = _elu(zp_ref[0, :, :nc] + zp_ref[1, :, :nc])
    m = jnp.max(h, axis=1, keepdims=True)
    lse = m + jnp.log(jnp.sum(jnp.exp(h - m), axis=1, keepdims=True))
    o_ref[...] = h - lse


def _final(zp, nc):
    C = zp.shape[2]
    return pl.pallas_call(
        _final_body,
        grid=(N // _BM,),
        in_specs=[pl.BlockSpec((NUM_CORES, _BM, C), lambda i: (0, i, 0))],
        out_specs=pl.BlockSpec((_BM, nc), lambda i: (i, 0)),
        out_shape=jax.ShapeDtypeStruct((N, nc), jnp.float32),
    )(zp)


def kernel(features, edge_index, W0, W1, W2):
    npad = E_PAD - E
    src = jnp.concatenate(
        [edge_index[0].astype(jnp.int32),
         N + (jnp.arange(npad, dtype=jnp.int32) % (N_PAD - N))]
    ).reshape(NW, NCHUNK, CHUNK)
    dst = jnp.concatenate(
        [edge_index[1].astype(jnp.int32),
         jnp.zeros((npad,), jnp.int32)]
    ).reshape(NW, NCHUNK, CHUNK)
    z128 = jnp.zeros((ROWS_PER_TILE, 128), jnp.float32)
    W2p = jnp.pad(W2, ((0, 0), (0, 128 - W2.shape[1])))

    hw = _mm(features, W0)
    zp = _make_agg(128)(hw, src, dst, z128)
    hw = _elu_mm(zp, W1)
    zp = _make_agg(128)(hw, src, dst, z128)
    hw = _elu_mm(zp, W2p)
    zp = _make_agg(128)(hw, src, dst, z128)
    return _final(zp, W2.shape[1])


# speedup vs baseline: 2.5419x; 2.5419x over previous
import functools

import jax
import jax.numpy as jnp
from jax import lax
from jax.experimental import pallas as pl
from jax.experimental.pallas import tpu as pltpu
from jax.experimental.pallas import tpu_sc as plsc

N = 10000
E = 160000
N_PAD = 10240
NUM_CORES = 2
NUM_SUBCORES = 16
NW = NUM_CORES * NUM_SUBCORES
CHUNK = 64
NCHUNK = 80
EPW = NCHUNK * CHUNK
E_PAD = NW * EPW
NPAIR = NCHUNK // 2
GC = 8
NGROUP = NCHUNK // GC
ROWS_PER_TILE = N_PAD // NUM_SUBCORES


@functools.cache
def _make_agg(H: int):
    mesh = plsc.VectorSubcoreMesh(core_axis_name="c", subcore_axis_name="s")

    @functools.partial(
        pl.kernel,
        out_type=jax.ShapeDtypeStruct((NUM_CORES, N_PAD, H), jnp.float32),
        mesh=mesh,
        compiler_params=pltpu.CompilerParams(use_tc_tiling_on_sc=True),
        scratch_types=[
            [pltpu.VMEM((GC, CHUNK), jnp.int32)] * 2,
            [pltpu.VMEM((GC, CHUNK), jnp.int32)] * 2,
            [pltpu.VMEM((CHUNK, H), jnp.float32)] * 4,
            [pltpu.SemaphoreType.DMA] * 4,
            [pltpu.SemaphoreType.DMA] * 4,
            [pltpu.SemaphoreType.DMA] * 4,
            pltpu.VMEM_SHARED((N_PAD, H), jnp.float32),
        ],
    )
    def agg(hw_hbm, src_hbm, dst_hbm, zeros_hbm, out_hbm,
            isrc, idst, gb, gsem, ssem, isem, acc):
        cid = lax.axis_index("c")
        sid = lax.axis_index("s")
        wid = cid * NUM_SUBCORES + sid
        base = sid * ROWS_PER_TILE
        pltpu.sync_copy(zeros_hbm, acc.at[pl.ds(base, ROWS_PER_TILE)])

        def idx_fetch(g, b):
            return (
                pltpu.make_async_copy(src_hbm.at[wid, pl.ds(g * GC, GC)],
                                      isrc[b], isem[2 * b]),
                pltpu.make_async_copy(dst_hbm.at[wid, pl.ds(g * GC, GC)],
                                      idst[b], isem[2 * b + 1]),
            )

        def gather(b, row, s):
            return pltpu.make_async_copy(hw_hbm.at[idst[b].at[row]], gb[s],
                                         gsem[s])

        def scatter(b, row, s):
            return pltpu.make_async_copy(gb[s], acc.at[isrc[b].at[row]],
                                         ssem[s])

        plsc.subcore_barrier()

        for d in idx_fetch(0, 0):
            d.start()

        def round_(r, t, gg):
            p = t % 2
            g0, g1 = 2 * p, 2 * p + 1
            s0, s1 = 2 - 2 * p, 3 - 2 * p
            b = 0 if t < 4 else 1
            rows = (2 * (t % 4), 2 * (t % 4) + 1)
            pb = 0 if (t - 1) % 8 < 4 else 1
            prows = (2 * ((t - 1) % 4), 2 * ((t - 1) % 4) + 1)
            db = 0 if (t - 2) % 8 < 4 else 1
            drows = (2 * ((t - 2) % 4), 2 * ((t - 2) % 4) + 1)

            @pl.when(r >= 2)
            def _():
                scatter(db, drows[0], g0).wait()
                scatter(db, drows[1], g1).wait()

            if t == 0:
                idx_fetch(2 * gg, 0)[0].wait()
                idx_fetch(2 * gg, 0)[1].wait()
            if t == 4:
                idx_fetch(2 * gg + 1, 1)[0].wait()
                idx_fetch(2 * gg + 1, 1)[1].wait()

            if t == 2:
                for d in idx_fetch(2 * gg + 1, 1):
                    d.start()
            if t == 6:
                @pl.when(2 * gg + 2 < NGROUP)
                def _():
                    for d in idx_fetch(2 * gg + 2, 0):
                        d.start()

            gather(b, rows[0], g0).start()
            gather(b, rows[1], g1).start()

            @pl.when(r >= 1)
            def _():
                gather(pb, prows[0], s0).wait()
                scatter(pb, prows[0], s0).start(add=True)
                gather(pb, prows[1], s1).wait()
                scatter(pb, prows[1], s1).start(add=True)

        def body(gg, carry):
            for t in range(8):
                round_(8 * gg + t, t, gg)
            return carry

        lax.fori_loop(0, NPAIR // 8, body, 0)
        scatter(1, 4, 0).wait()
        scatter(1, 5, 1).wait()
        gather(1, 6, 2).wait()
        scatter(1, 6, 2).start(add=True)
        gather(1, 7, 3).wait()
        scatter(1, 7, 3).start(add=True)
        scatter(1, 6, 2).wait()
        scatter(1, 7, 3).wait()

        plsc.subcore_barrier()
        pltpu.sync_copy(acc.at[pl.ds(base, ROWS_PER_TILE)],
                        out_hbm.at[cid, pl.ds(base, ROWS_PER_TILE)])

    return agg


_BM = 2000


def _mm_body(x_ref, w_ref, o_ref):
    o_ref[...] = jnp.dot(x_ref[...], w_ref[...],
                         preferred_element_type=jnp.float32)


def _mm(x, W):
    M, K = x.shape
    H = W.shape[1]
    return pl.pallas_call(
        _mm_body,
        grid=(M // _BM,),
        in_specs=[pl.BlockSpec((_BM, K), lambda i: (i, 0)),
                  pl.BlockSpec((K, H), lambda i: (0, 0))],
        out_specs=pl.BlockSpec((_BM, H), lambda i: (i, 0)),
        out_shape=jax.ShapeDtypeStruct((M, H), jnp.float32),
    )(x, W)


def _elu(z):
    return jnp.where(z > 0, z, jnp.exp(z) - 1.0)


def _elu_mm_body(zp_ref, w_ref, o_ref):
    h = _elu(zp_ref[0] + zp_ref[1])
    o_ref[...] = jnp.dot(h, w_ref[...], preferred_element_type=jnp.float32)


def _elu_mm(zp, W):
    K = zp.shape[2]
    H = W.shape[1]
    return pl.pallas_call(
        _elu_mm_body,
        grid=(N // _BM,),
        in_specs=[pl.BlockSpec((NUM_CORES, _BM, K), lambda i: (0, i, 0)),
                  pl.BlockSpec((K, H), lambda i: (0, 0))],
        out_specs=pl.BlockSpec((_BM, H), lambda i: (i, 0)),
        out_shape=jax.ShapeDtypeStruct((N, H), jnp.float32),
    )(zp, W)


def _final_body(zp_ref, o_ref):
    nc = o_ref.shape[1]
    h = _elu(zp_ref[0, :, :nc] + zp_ref[1, :, :nc])
    m = jnp.max(h, axis=1, keepdims=True)
    lse = m + jnp.log(jnp.sum(jnp.exp(h - m), axis=1, keepdims=True))
    o_ref[...] = h - lse


def _final(zp, nc):
    C = zp.shape[2]
    return pl.pallas_call(
        _final_body,
        grid=(N // _BM,),
        in_specs=[pl.BlockSpec((NUM_CORES, _BM, C), lambda i: (0, i, 0))],
        out_specs=pl.BlockSpec((_BM, nc), lambda i: (i, 0)),
        out_shape=jax.ShapeDtypeStruct((N, nc), jnp.float32),
    )(zp)


def kernel(features, edge_index, W0, W1, W2):
    npad = E_PAD - E
    src = jnp.concatenate(
        [edge_index[0].astype(jnp.int32),
         N + (jnp.arange(npad, dtype=jnp.int32) % (N_PAD - N))]
    ).reshape(NW, NCHUNK, CHUNK)
    dst = jnp.concatenate(
        [edge_index[1].astype(jnp.int32),
         jnp.arange(npad, dtype=jnp.int32) % N]
    ).reshape(NW, NCHUNK, CHUNK)
    z128 = jnp.zeros((ROWS_PER_TILE, 128), jnp.float32)
    W2p = jnp.pad(W2, ((0, 0), (0, 128 - W2.shape[1])))

    hw = _mm(features, W0)
    zp = _make_agg(128)(hw, src, dst, z128)
    hw = _elu_mm(zp, W1)
    zp = _make_agg(128)(hw, src, dst, z128)
    hw = _elu_mm(zp, W2p)
    zp = _make_agg(128)(hw, src, dst, z128)
    return _final(zp, W2.shape[1])


# speedup vs baseline: 2.5713x; 1.0116x over previous
import functools

import jax
import jax.numpy as jnp
from jax import lax
from jax.experimental import pallas as pl
from jax.experimental.pallas import tpu as pltpu
from jax.experimental.pallas import tpu_sc as plsc

N = 10000
E = 160000
N_PAD = 10240
NUM_CORES = 2
NUM_SUBCORES = 16
NW = NUM_CORES * NUM_SUBCORES
CHUNK = 80
NCHUNK = 64
EPW = NCHUNK * CHUNK
E_PAD = NW * EPW
NPAIR = NCHUNK // 2
GC = 8
NGROUP = NCHUNK // GC
ROWS_PER_TILE = N_PAD // NUM_SUBCORES


@functools.cache
def _make_agg(H: int):
    mesh = plsc.VectorSubcoreMesh(core_axis_name="c", subcore_axis_name="s")

    @functools.partial(
        pl.kernel,
        out_type=jax.ShapeDtypeStruct((NUM_CORES, N_PAD, H), jnp.float32),
        mesh=mesh,
        compiler_params=pltpu.CompilerParams(use_tc_tiling_on_sc=True),
        scratch_types=[
            [pltpu.VMEM((GC, CHUNK), jnp.int32)] * 2,
            [pltpu.VMEM((GC, CHUNK), jnp.int32)] * 2,
            [pltpu.VMEM((CHUNK, H), jnp.float32)] * 4,
            [pltpu.SemaphoreType.DMA] * 4,
            [pltpu.SemaphoreType.DMA] * 4,
            [pltpu.SemaphoreType.DMA] * 4,
            pltpu.VMEM_SHARED((N_PAD, H), jnp.float32),
        ],
    )
    def agg(hw_hbm, edges_hbm, zeros_hbm, out_hbm,
            isrc, idst, gb, gsem, ssem, isem, acc):
        cid = lax.axis_index("c")
        sid = lax.axis_index("s")
        wid = cid * NUM_SUBCORES + sid
        base = sid * ROWS_PER_TILE
        pltpu.sync_copy(zeros_hbm, acc.at[pl.ds(base, ROWS_PER_TILE)])

        def idx_fetch(g, b):
            return (
                pltpu.make_async_copy(
                    edges_hbm.at[0, wid, pl.ds(g * GC, GC)],
                    isrc[b], isem[2 * b]),
                pltpu.make_async_copy(
                    edges_hbm.at[1, wid, pl.ds(g * GC, GC)],
                    idst[b], isem[2 * b + 1]),
            )

        def gather(b, row, s):
            return pltpu.make_async_copy(hw_hbm.at[idst[b].at[row]], gb[s],
                                         gsem[s])

        def scatter(b, row, s):
            return pltpu.make_async_copy(gb[s], acc.at[isrc[b].at[row]],
                                         ssem[s])

        plsc.subcore_barrier()

        for d in idx_fetch(0, 0):
            d.start()

        def round_(r, t, gg):
            p = t % 2
            g0, g1 = 2 * p, 2 * p + 1
            s0, s1 = 2 - 2 * p, 3 - 2 * p
            b = 0 if t < 4 else 1
            rows = (2 * (t % 4), 2 * (t % 4) + 1)
            pb = 0 if (t - 1) % 8 < 4 else 1
            prows = (2 * ((t - 1) % 4), 2 * ((t - 1) % 4) + 1)
            db = 0 if (t - 2) % 8 < 4 else 1
            drows = (2 * ((t - 2) % 4), 2 * ((t - 2) % 4) + 1)

            @pl.when(r >= 2)
            def _():
                scatter(db, drows[0], g0).wait()
                scatter(db, drows[1], g1).wait()

            if t == 0:
                idx_fetch(2 * gg, 0)[0].wait()
                idx_fetch(2 * gg, 0)[1].wait()
            if t == 4:
                idx_fetch(2 * gg + 1, 1)[0].wait()
                idx_fetch(2 * gg + 1, 1)[1].wait()

            if t == 2:
                for d in idx_fetch(2 * gg + 1, 1):
                    d.start()
            if t == 6:
                @pl.when(2 * gg + 2 < NGROUP)
                def _():
                    for d in idx_fetch(2 * gg + 2, 0):
                        d.start()

            gather(b, rows[0], g0).start()
            gather(b, rows[1], g1).start()

            @pl.when(r >= 1)
            def _():
                gather(pb, prows[0], s0).wait()
                scatter(pb, prows[0], s0).start(add=True)
                gather(pb, prows[1], s1).wait()
                scatter(pb, prows[1], s1).start(add=True)

        def body(gg, carry):
            for t in range(8):
                round_(8 * gg + t, t, gg)
            return carry

        lax.fori_loop(0, NPAIR // 8, body, 0)
        scatter(1, 4, 0).wait()
        scatter(1, 5, 1).wait()
        gather(1, 6, 2).wait()
        scatter(1, 6, 2).start(add=True)
        gather(1, 7, 3).wait()
        scatter(1, 7, 3).start(add=True)
        scatter(1, 6, 2).wait()
        scatter(1, 7, 3).wait()

        plsc.subcore_barrier()
        pltpu.sync_copy(acc.at[pl.ds(base, ROWS_PER_TILE)],
                        out_hbm.at[cid, pl.ds(base, ROWS_PER_TILE)])

    return agg


_BM = 2000


def _mm_body(x_ref, w_ref, o_ref):
    o_ref[...] = jnp.dot(x_ref[...], w_ref[...],
                         preferred_element_type=jnp.float32)


def _mm(x, W):
    M, K = x.shape
    H = W.shape[1]
    return pl.pallas_call(
        _mm_body,
        grid=(M // _BM,),
        in_specs=[pl.BlockSpec((_BM, K), lambda i: (i, 0)),
                  pl.BlockSpec((K, H), lambda i: (0, 0))],
        out_specs=pl.BlockSpec((_BM, H), lambda i: (i, 0)),
        out_shape=jax.ShapeDtypeStruct((M, H), jnp.float32),
    )(x, W)


def _elu(z):
    return jnp.where(z > 0, z, jnp.exp(z) - 1.0)


def _elu_mm_body(zp_ref, w_ref, o_ref):
    h = _elu(zp_ref[0] + zp_ref[1])
    o_ref[...] = jnp.dot(h, w_ref[...], preferred_element_type=jnp.float32)


def _elu_mm(zp, W):
    K = zp.shape[2]
    H = W.shape[1]
    return pl.pallas_call(
        _elu_mm_body,
        grid=(N // _BM,),
        in_specs=[pl.BlockSpec((NUM_CORES, _BM, K), lambda i: (0, i, 0)),
                  pl.BlockSpec((K, H), lambda i: (0, 0))],
        out_specs=pl.BlockSpec((_BM, H), lambda i: (i, 0)),
        out_shape=jax.ShapeDtypeStruct((N, H), jnp.float32),
    )(zp, W)


def _final_body(zp_ref, o_ref):
    nc = o_ref.shape[1]
    h = _elu(zp_ref[0, :, :nc] + zp_ref[1, :, :nc])
    m = jnp.max(h, axis=1, keepdims=True)
    lse = m + jnp.log(jnp.sum(jnp.exp(h - m), axis=1, keepdims=True))
    o_ref[...] = h - lse


def _final(zp, nc):
    C = zp.shape[2]
    return pl.pallas_call(
        _final_body,
        grid=(N // _BM,),
        in_specs=[pl.BlockSpec((NUM_CORES, _BM, C), lambda i: (0, i, 0))],
        out_specs=pl.BlockSpec((_BM, nc), lambda i: (i, 0)),
        out_shape=jax.ShapeDtypeStruct((N, nc), jnp.float32),
    )(zp)


def kernel(features, edge_index, W0, W1, W2):
    npad = E_PAD - E
    ar = jnp.arange(npad, dtype=jnp.int32)
    dummies = jnp.stack([N + ar % (N_PAD - N), ar % N])
    edges = jnp.concatenate(
        [edge_index.astype(jnp.int32), dummies], axis=1
    ).reshape(2, NW, NCHUNK, CHUNK)
    z128 = jnp.zeros((ROWS_PER_TILE, 128), jnp.float32)
    W2p = jnp.pad(W2, ((0, 0), (0, 128 - W2.shape[1])))

    hw = _mm(features, W0)
    zp = _make_agg(128)(hw, edges, z128)
    hw = _elu_mm(zp, W1)
    zp = _make_agg(128)(hw, edges, z128)
    hw = _elu_mm(zp, W2p)
    zp = _make_agg(128)(hw, edges, z128)
    return _final(zp, W2.shape[1])
